# baseline (speedup 1.0000x reference)
import functools

import jax
import jax.numpy as jnp
from jax import lax
from jax.experimental import pallas as pl
from jax.experimental.pallas import tpu as pltpu
from jax.experimental.pallas import tpu_sc as plsc

_K = 40
_INT_MIN = -(2**31)
_NB = 64
_NR = 384
_RL = 384
_NW = 32
_BPW = _NB // _NW
_WAVE = 48
_NRV = _NR // 16
_CAP = 512
_CAPV = _CAP // 16
_OUT = 128


def _softplus(x):
    return jnp.log(jnp.exp(x) + 1.0)


def _sort_key(x):
    u = lax.bitcast_convert_type(x, jnp.int32)
    return u ^ jnp.right_shift(u, 31) & jnp.int32(0x7FFFFFFF)


def _key_to_val(k):
    return lax.bitcast_convert_type(k ^ jnp.right_shift(k, 31)
                                    & jnp.int32(0x7FFFFFFF), jnp.float32)


def _stage_a(y_ref, m_ref, rm_ref, ps_ref):
    y = y_ref[0]
    m = m_ref[0]
    ps_num = jnp.sum(jnp.where(m, _softplus(-y), 0.0))
    ps_den = jnp.sum(m.astype(jnp.float32))
    rm_ref[0, 0] = jnp.max(jnp.where(m, -jnp.inf, y), axis=1)
    lane = lax.broadcasted_iota(jnp.int32, (1, 1, 128), 2)
    ps_ref[...] = jnp.where(lane == 0, ps_num, jnp.where(lane == 1, ps_den, 0.0))


def _b16(s, dtype=jnp.int32):
    return jnp.full((16,), s, dtype)


def _count40_key(read_vreg, nv):
    def bit_step(i, tu):
        cand = tu | jnp.left_shift(jnp.int32(1), jnp.int32(31) - i)
        thr = _b16(cand ^ jnp.int32(_INT_MIN))
        cnt_v = _b16(0)
        for j in range(nv):
            cnt_v = cnt_v + jnp.where(read_vreg(j) >= thr, 1, 0)
        cnt = jnp.sum(cnt_v)
        return jnp.where(cnt >= _K, cand, tu)

    tu = lax.fori_loop(0, 32, bit_step, jnp.int32(0), unroll=False)
    return tu ^ jnp.int32(_INT_MIN)


def _val_of_key(read_val, nv, tkey):
    tkeyv = _b16(tkey)
    acc = jnp.full((16,), -jnp.inf, jnp.float32)
    for j in range(nv):
        v = read_val(j)
        acc = jnp.where(_sort_key(v) == tkeyv, v, acc)
    return jnp.max(acc)


def _sc_body(y_hbm, m_hbm, rm_hbm, out_hbm, rmf_v, rmk_v, idx_v, idxm_v,
             rows_v, rowm_v, b_v, b2_v, sem, msem):
    wid = lax.axis_index("s") * 2 + lax.axis_index("c")
    neg_inf = jnp.full((16,), -jnp.inf, jnp.float32)
    lane16 = lax.iota(jnp.int32, 16)

    def reselect(c, tkey, tval):
        t40 = _count40_key(lambda j: _sort_key(b_v[pl.ds(j * 16, 16)]),
                           _CAPV)
        t40val = _val_of_key(lambda j: b_v[pl.ds(j * 16, 16)], _CAPV, t40)
        t40v = _b16(t40)
        for j in range(_CAPV):
            b2_v[pl.ds(j * 16, 16)] = neg_inf

        def compact(j, off):
            v = b_v[pl.ds(j * 16, 16)]
            m = _sort_key(v) > t40v
            plsc.store_compressed(b2_v.at[pl.ds(off, 16)], v, mask=m)
            return off + jnp.sum(jnp.where(m, 1, 0))
        c2 = lax.fori_loop(0, _CAPV, compact, jnp.int32(0), unroll=False)
        for j in range(_CAPV):
            b_v[pl.ds(j * 16, 16)] = b2_v[pl.ds(j * 16, 16)]
        return c2, t40, t40val

    def keep(c, tkey, tval):
        return c, tkey, tval

    def process_batch(b):
        base_row = b * _NR
        pltpu.sync_copy(rm_hbm.at[b], rmf_v)
        for j in range(_NRV):
            rmk_v[pl.ds(j * 16, 16)] = _sort_key(rmf_v[pl.ds(j * 16, 16)])
        tkey = _count40_key(lambda j: rmk_v[pl.ds(j * 16, 16)], _NRV)
        tval = _val_of_key(lambda j: rmf_v[pl.ds(j * 16, 16)], _NRV, tkey)

        t1v = _b16(tkey)
        for j in range(_NRV + 1):
            rid = lane16 + _b16(j * 16 + base_row)
            idx_v[pl.ds(j * 16, 16)] = rid
            idxm_v[pl.ds(j * 16, 16)] = jnp.right_shift(rid, 2)
        nr_v = _b16(0)
        for j in range(_NRV):
            m = rmk_v[pl.ds(j * 16, 16)] >= t1v
            mi = jnp.where(m, 1, 0)
            pos = nr_v + plsc.cumsum(mi) - mi
            rid = lane16 + _b16(j * 16 + base_row)
            plsc.store_scatter(idx_v, [pos], rid, mask=m)
            plsc.store_scatter(idxm_v, [pos], jnp.right_shift(rid, 2),
                               mask=m)
            nr_v = nr_v + plsc.all_reduce_population_count(m)
        n_rows = jnp.max(nr_v)

        for j in range(_CAPV):
            b_v[pl.ds(j * 16, 16)] = neg_inf

        n_waves = (n_rows + (_WAVE - 1)) // _WAVE

        word_idx = jnp.right_shift(lane16, 2)
        bshift = (lane16 & jnp.int32(3)) * jnp.int32(8)

        def wave(w, carry):
            c, tkey, tval = carry
            ycp = pltpu.async_copy(
                y_hbm.at[idx_v.at[pl.ds(w * _WAVE, _WAVE)]], rows_v, sem)
            mcp = pltpu.async_copy(
                m_hbm.at[idxm_v.at[pl.ds(w * _WAVE, _WAVE)]], rowm_v, msem)
            ycp.wait()
            mcp.wait()
            rows_here = jnp.minimum(jnp.int32(_WAVE), n_rows - w * _WAVE)

            def row(r, carry):
                c, tkey, tval = carry
                c, tkey, tval = lax.cond(c > _CAP - _RL, reselect, keep,
                                         c, tkey, tval)
                tkeyv = _b16(tkey)
                rv = _b16(r)
                gid = idx_v[pl.ds(w * _WAVE + r, 16)][0]
                woff = (gid & jnp.int32(3)) * jnp.int32(_RL // 4)

                def vreg(j, cv):
                    v = rows_v[r, pl.ds(j * 16, 16)]
                    mw = plsc.load_gather(
                        rowm_v, [rv, _b16(woff + j * 4) + word_idx])
                    mbyte = jnp.right_shift(mw, bshift) & jnp.int32(0xFF)
                    m = jnp.logical_and(_sort_key(v) > tkeyv, mbyte == 0)
                    mi = jnp.where(m, 1, 0)
                    pos = cv + plsc.cumsum(mi) - mi
                    plsc.store_scatter(b_v, [pos], v, mask=m)
                    return cv + plsc.all_reduce_population_count(m)
                cv = lax.fori_loop(0, _RL // 16, vreg, _b16(c),
                                   unroll=False)
                return jnp.max(cv), tkey, tval
            return lax.fori_loop(0, rows_here, row, (c, tkey, tval),
                                 unroll=False)

        c, tkey, tval = lax.fori_loop(0, n_waves, wave,
                                      (jnp.int32(0), tkey, tval),
                                      unroll=False)
        c, tkey, tval = lax.cond(c > _OUT, reselect, keep, c, tkey, tval)
        tvalv = _b16(tval, jnp.float32)
        cv = _b16(c)
        kv = _b16(_K)
        for j in range(_OUT // 16):
            l = lane16 + _b16(j * 16)
            v = b_v[pl.ds(j * 16, 16)]
            v = jnp.where(l < cv, v, jnp.where(l < kv, tvalv, neg_inf))
            b2_v[pl.ds(j * 16, 16)] = v
        pltpu.sync_copy(b2_v.at[pl.ds(0, _OUT)], out_hbm.at[b])

    for bi in range(_BPW):
        process_batch(wid * _BPW + bi)


def _make_sc_kernel():
    return functools.partial(
        pl.kernel,
        out_type=jax.ShapeDtypeStruct((_NB, _OUT), jnp.float32),
        mesh=plsc.VectorSubcoreMesh(core_axis_name="c", subcore_axis_name="s",
                                    num_cores=2, num_subcores=16),
        compiler_params=pltpu.CompilerParams(needs_layout_passes=False),
        scratch_types=[
            pltpu.VMEM((_NR,), jnp.float32),
            pltpu.VMEM((_NR,), jnp.int32),
            pltpu.VMEM((_NR + 16,), jnp.int32),
            pltpu.VMEM((_NR + 16,), jnp.int32),
            pltpu.VMEM((_WAVE, _RL), jnp.float32),
            pltpu.VMEM((_WAVE, _RL), jnp.int32),
            pltpu.VMEM((_CAP,), jnp.float32),
            pltpu.VMEM((_CAP,), jnp.float32),
            pltpu.SemaphoreType.DMA,
            pltpu.SemaphoreType.DMA,
        ],
    )(_sc_body)


def _stage_c(cand_ref, ps_ref, out_ref):
    cand = cand_ref[...]
    key = _sort_key(cand)

    def bit_step(i, tu):
        candbit = tu | jnp.left_shift(jnp.int32(1), jnp.int32(31) - i)
        thr = candbit ^ jnp.int32(_INT_MIN)
        cnt = jnp.sum((key >= thr).astype(jnp.int32), axis=1, keepdims=True)
        return jnp.where(cnt >= _K, candbit, tu)

    tu = lax.fori_loop(0, 32, bit_step, jnp.zeros((_NB, 1), jnp.int32))
    t = tu ^ jnp.int32(_INT_MIN)
    gt = key > t
    cnt_gt = jnp.sum(gt.astype(jnp.float32), axis=1, keepdims=True)
    sum_gt = jnp.sum(jnp.where(gt, _softplus(cand), 0.0), axis=1,
                     keepdims=True)
    ns_b = sum_gt + (jnp.float32(_K) - cnt_gt) * _softplus(_key_to_val(t))
    ns = jnp.sum(ns_b) / jnp.float32(_NB * _K)
    ps = ps_ref[:, 0, 0:1] / ps_ref[:, 0, 1:2]
    out_ref[...] = jnp.broadcast_to(ns + ps, (_NB, 128))


def kernel(ypred, truthMask):
    rm, ps = pl.pallas_call(
        _stage_a,
        grid=(_NB,),
        in_specs=[
            pl.BlockSpec((1, _NR, _RL), lambda i: (i, 0, 0)),
            pl.BlockSpec((1, _NR, _RL), lambda i: (i, 0, 0)),
        ],
        out_specs=[
            pl.BlockSpec((1, 1, _NR), lambda i: (i, 0, 0)),
            pl.BlockSpec((1, 1, 128), lambda i: (i, 0, 0)),
        ],
        out_shape=[
            jax.ShapeDtypeStruct((_NB, 1, _NR), jnp.float32),
            jax.ShapeDtypeStruct((_NB, 1, 128), jnp.float32),
        ],
    )(ypred, truthMask)

    m32 = lax.bitcast_convert_type(
        truthMask.reshape(_NB * _NR, _RL // 4, 4).astype(jnp.uint8),
        jnp.int32).reshape(_NB * _NR // 4, _RL)
    cand = _make_sc_kernel()(ypred.reshape(_NB * _NR, _RL), m32,
                             rm.reshape(_NB, _NR))

    out = pl.pallas_call(
        _stage_c,
        in_specs=[
            pl.BlockSpec((_NB, _OUT), lambda: (0, 0)),
            pl.BlockSpec((_NB, 1, 128), lambda: (0, 0, 0)),
        ],
        out_specs=pl.BlockSpec((_NB, 128), lambda: (0, 0)),
        out_shape=jax.ShapeDtypeStruct((_NB, 128), jnp.float32),
    )(cand, ps)
    return out[:, 0]


# speedup vs baseline: 1.8945x; 1.8945x over previous
import functools

import jax
import jax.numpy as jnp
from jax import lax
from jax.experimental import pallas as pl
from jax.experimental.pallas import tpu as pltpu
from jax.experimental.pallas import tpu_sc as plsc

_K = 40
_INT_MIN = -(2**31)
_NB = 64
_NR = 384
_RL = 384
_NW = 32
_BPW = _NB // _NW
_WAVE = 48
_NRV = _NR // 16
_CAP = 512
_CAPV = _CAP // 16
_OUT = 128


def _softplus(x):
    return jnp.log(jnp.exp(x) + 1.0)


def _sort_key(x):
    u = lax.bitcast_convert_type(x, jnp.int32)
    return u ^ jnp.right_shift(u, 31) & jnp.int32(0x7FFFFFFF)


def _key_to_val(k):
    return lax.bitcast_convert_type(k ^ jnp.right_shift(k, 31)
                                    & jnp.int32(0x7FFFFFFF), jnp.float32)


def _stage_a(y_ref, m_ref, x_ref, rm_ref, ps_ref):
    y = y_ref[0]
    m = m_ref[0]
    ps_num = jnp.sum(jnp.where(m, _softplus(-y), 0.0))
    ps_den = jnp.sum(m.astype(jnp.float32))
    x = jnp.where(m, -jnp.inf, y)
    x_ref[0] = x
    rm_ref[0, 0] = jnp.max(x, axis=1)
    lane = lax.broadcasted_iota(jnp.int32, (1, 1, 128), 2)
    ps_ref[...] = jnp.where(lane == 0, ps_num, jnp.where(lane == 1, ps_den, 0.0))


def _b16(s, dtype=jnp.int32):
    return jnp.full((16,), s, dtype)


def _count40_key(read_vreg, nv):
    def bit_step(i, tu):
        cand = tu | jnp.left_shift(jnp.int32(1), jnp.int32(31) - i)
        thr = _b16(cand ^ jnp.int32(_INT_MIN))
        cnt_v = _b16(0)
        for j in range(nv):
            cnt_v = cnt_v + jnp.where(read_vreg(j) >= thr, 1, 0)
        cnt = jnp.sum(cnt_v)
        return jnp.where(cnt >= _K, cand, tu)

    tu = lax.fori_loop(0, 32, bit_step, jnp.int32(0), unroll=False)
    return tu ^ jnp.int32(_INT_MIN)


def _val_of_key(read_val, nv, tkey):
    tkeyv = _b16(tkey)
    acc = jnp.full((16,), -jnp.inf, jnp.float32)
    for j in range(nv):
        v = read_val(j)
        acc = jnp.where(_sort_key(v) == tkeyv, v, acc)
    return jnp.max(acc)


def _sc_body(x_hbm, rm_hbm, out_hbm, rmf_v, rmk_v, idx_v, rows_v, b_v,
             b2_v, sem):
    wid = lax.axis_index("s") * 2 + lax.axis_index("c")
    neg_inf = jnp.full((16,), -jnp.inf, jnp.float32)
    lane16 = lax.iota(jnp.int32, 16)

    def reselect(c, tkey, tval):
        t40 = _count40_key(lambda j: _sort_key(b_v[pl.ds(j * 16, 16)]),
                           _CAPV)
        t40val = _val_of_key(lambda j: b_v[pl.ds(j * 16, 16)], _CAPV, t40)
        t40v = _b16(t40)
        for j in range(_CAPV):
            b2_v[pl.ds(j * 16, 16)] = neg_inf

        def compact(j, off):
            v = b_v[pl.ds(j * 16, 16)]
            m = _sort_key(v) > t40v
            plsc.store_compressed(b2_v.at[pl.ds(off, 16)], v, mask=m)
            return off + jnp.sum(jnp.where(m, 1, 0))
        c2 = lax.fori_loop(0, _CAPV, compact, jnp.int32(0), unroll=False)
        for j in range(_CAPV):
            b_v[pl.ds(j * 16, 16)] = b2_v[pl.ds(j * 16, 16)]
        return c2, t40, t40val

    def keep(c, tkey, tval):
        return c, tkey, tval

    def process_batch(b):
        base_row = b * _NR
        pltpu.sync_copy(rm_hbm.at[b], rmf_v)
        for j in range(_NRV):
            rmk_v[pl.ds(j * 16, 16)] = _sort_key(rmf_v[pl.ds(j * 16, 16)])
        tkey = _count40_key(lambda j: rmk_v[pl.ds(j * 16, 16)], _NRV)
        tval = _val_of_key(lambda j: rmf_v[pl.ds(j * 16, 16)], _NRV, tkey)

        t1v = _b16(tkey)
        for j in range(_NRV + 1):
            idx_v[pl.ds(j * 16, 16)] = lane16 + _b16(j * 16 + base_row)
        nr_v = _b16(0)
        for j in range(_NRV):
            m = rmk_v[pl.ds(j * 16, 16)] >= t1v
            mi = jnp.where(m, 1, 0)
            pos = nr_v + plsc.cumsum(mi) - mi
            plsc.store_scatter(idx_v, [pos],
                               lane16 + _b16(j * 16 + base_row), mask=m)
            nr_v = nr_v + plsc.all_reduce_population_count(m)
        n_rows = jnp.max(nr_v)

        for j in range(_CAPV):
            b_v[pl.ds(j * 16, 16)] = neg_inf

        n_waves = (n_rows + (_WAVE - 1)) // _WAVE

        def wave(w, carry):
            c, tkey, tval = carry
            pltpu.async_copy(x_hbm.at[idx_v.at[pl.ds(w * _WAVE, _WAVE)]],
                             rows_v, sem).wait()
            rows_here = jnp.minimum(jnp.int32(_WAVE), n_rows - w * _WAVE)

            def row(r, carry):
                c, tkey, tval = carry
                c, tkey, tval = lax.cond(c > _CAP - _RL, reselect, keep,
                                         c, tkey, tval)
                tkeyv = _b16(tkey)

                def vreg(j, cv):
                    v = rows_v[r, pl.ds(j * 16, 16)]
                    m = _sort_key(v) > tkeyv
                    mi = jnp.where(m, 1, 0)
                    pos = cv + plsc.cumsum(mi) - mi
                    plsc.store_scatter(b_v, [pos], v, mask=m)
                    return cv + plsc.all_reduce_population_count(m)
                cv = lax.fori_loop(0, _RL // 16, vreg, _b16(c),
                                   unroll=True)
                return jnp.max(cv), tkey, tval
            return lax.fori_loop(0, rows_here, row, (c, tkey, tval),
                                 unroll=False)

        c, tkey, tval = lax.fori_loop(0, n_waves, wave,
                                      (jnp.int32(0), tkey, tval),
                                      unroll=False)
        c, tkey, tval = lax.cond(c > _OUT, reselect, keep, c, tkey, tval)
        tvalv = _b16(tval, jnp.float32)
        cv = _b16(c)
        kv = _b16(_K)
        for j in range(_OUT // 16):
            l = lane16 + _b16(j * 16)
            v = b_v[pl.ds(j * 16, 16)]
            v = jnp.where(l < cv, v, jnp.where(l < kv, tvalv, neg_inf))
            b2_v[pl.ds(j * 16, 16)] = v
        pltpu.sync_copy(b2_v.at[pl.ds(0, _OUT)], out_hbm.at[b])

    for bi in range(_BPW):
        process_batch(wid * _BPW + bi)


def _make_sc_kernel():
    return functools.partial(
        pl.kernel,
        out_type=jax.ShapeDtypeStruct((_NB, _OUT), jnp.float32),
        mesh=plsc.VectorSubcoreMesh(core_axis_name="c", subcore_axis_name="s",
                                    num_cores=2, num_subcores=16),
        compiler_params=pltpu.CompilerParams(needs_layout_passes=False),
        scratch_types=[
            pltpu.VMEM((_NR,), jnp.float32),
            pltpu.VMEM((_NR,), jnp.int32),
            pltpu.VMEM((_NR + 16,), jnp.int32),
            pltpu.VMEM((_WAVE, _RL), jnp.float32),
            pltpu.VMEM((_CAP,), jnp.float32),
            pltpu.VMEM((_CAP,), jnp.float32),
            pltpu.SemaphoreType.DMA,
        ],
    )(_sc_body)


def _stage_c(cand_ref, ps_ref, out_ref):
    cand = cand_ref[...]
    key = _sort_key(cand)

    def bit_step(i, tu):
        candbit = tu | jnp.left_shift(jnp.int32(1), jnp.int32(31) - i)
        thr = candbit ^ jnp.int32(_INT_MIN)
        cnt = jnp.sum((key >= thr).astype(jnp.int32), axis=1, keepdims=True)
        return jnp.where(cnt >= _K, candbit, tu)

    tu = lax.fori_loop(0, 32, bit_step, jnp.zeros((_NB, 1), jnp.int32))
    t = tu ^ jnp.int32(_INT_MIN)
    gt = key > t
    cnt_gt = jnp.sum(gt.astype(jnp.float32), axis=1, keepdims=True)
    sum_gt = jnp.sum(jnp.where(gt, _softplus(cand), 0.0), axis=1,
                     keepdims=True)
    ns_b = sum_gt + (jnp.float32(_K) - cnt_gt) * _softplus(_key_to_val(t))
    ns = jnp.sum(ns_b) / jnp.float32(_NB * _K)
    ps = ps_ref[:, 0, 0:1] / ps_ref[:, 0, 1:2]
    out_ref[...] = jnp.broadcast_to(ns + ps, (_NB, 128))


def kernel(ypred, truthMask):
    x, rm, ps = pl.pallas_call(
        _stage_a,
        grid=(_NB,),
        in_specs=[
            pl.BlockSpec((1, _NR, _RL), lambda i: (i, 0, 0)),
            pl.BlockSpec((1, _NR, _RL), lambda i: (i, 0, 0)),
        ],
        out_specs=[
            pl.BlockSpec((1, _NR, _RL), lambda i: (i, 0, 0)),
            pl.BlockSpec((1, 1, _NR), lambda i: (i, 0, 0)),
            pl.BlockSpec((1, 1, 128), lambda i: (i, 0, 0)),
        ],
        out_shape=[
            jax.ShapeDtypeStruct((_NB, _NR, _RL), jnp.float32),
            jax.ShapeDtypeStruct((_NB, 1, _NR), jnp.float32),
            jax.ShapeDtypeStruct((_NB, 1, 128), jnp.float32),
        ],
    )(ypred, truthMask)

    cand = _make_sc_kernel()(x.reshape(_NB * _NR, _RL), rm.reshape(_NB, _NR))

    out = pl.pallas_call(
        _stage_c,
        in_specs=[
            pl.BlockSpec((_NB, _OUT), lambda: (0, 0)),
            pl.BlockSpec((_NB, 1, 128), lambda: (0, 0, 0)),
        ],
        out_specs=pl.BlockSpec((_NB, 128), lambda: (0, 0)),
        out_shape=jax.ShapeDtypeStruct((_NB, 128), jnp.float32),
    )(cand, ps)
    return out[:, 0]


# speedup vs baseline: 2.1034x; 1.1103x over previous
import functools

import jax
import jax.numpy as jnp
from jax import lax
from jax.experimental import pallas as pl
from jax.experimental.pallas import tpu as pltpu
from jax.experimental.pallas import tpu_sc as plsc

_K = 40
_INT_MIN = -(2**31)
_NB = 64
_NR = 384
_RL = 384
_NW = 32
_BPW = _NB // _NW
_WAVE = 48
_NRV = _NR // 16
_CAP = 512
_CAPV = _CAP // 16
_OUT = 128


def _softplus(x):
    return jnp.log(jnp.exp(x) + 1.0)


def _sort_key(x):
    u = lax.bitcast_convert_type(x, jnp.int32)
    return u ^ jnp.right_shift(u, 31) & jnp.int32(0x7FFFFFFF)


def _key_to_val(k):
    return lax.bitcast_convert_type(k ^ jnp.right_shift(k, 31)
                                    & jnp.int32(0x7FFFFFFF), jnp.float32)


def _stage_a(y_ref, m_ref, x_ref, rm_ref, ps_ref):
    y = y_ref[0]
    m = m_ref[0]
    ps_num = jnp.sum(jnp.where(m, _softplus(-y), 0.0))
    ps_den = jnp.sum(m.astype(jnp.float32))
    x = jnp.where(m, -jnp.inf, y)
    x_ref[0] = x
    rm_ref[0, 0] = jnp.max(x, axis=1)
    lane = lax.broadcasted_iota(jnp.int32, (1, 1, 128), 2)
    ps_ref[...] = jnp.where(lane == 0, ps_num, jnp.where(lane == 1, ps_den, 0.0))


def _b16(s, dtype=jnp.int32):
    return jnp.full((16,), s, dtype)


def _count40_key(read_vreg, nv):
    def bit_step(i, tu):
        cand = tu | jnp.left_shift(jnp.int32(1), jnp.int32(31) - i)
        thr = _b16(cand ^ jnp.int32(_INT_MIN))
        cnt_v = _b16(0)
        for j in range(nv):
            cnt_v = cnt_v + jnp.where(read_vreg(j) >= thr, 1, 0)
        cnt = jnp.sum(cnt_v)
        return jnp.where(cnt >= _K, cand, tu)

    tu = lax.fori_loop(0, 32, bit_step, jnp.int32(0), unroll=False)
    return tu ^ jnp.int32(_INT_MIN)


def _val_of_key(read_val, nv, tkey):
    tkeyv = _b16(tkey)
    acc = jnp.full((16,), -jnp.inf, jnp.float32)
    for j in range(nv):
        v = read_val(j)
        acc = jnp.where(_sort_key(v) == tkeyv, v, acc)
    return jnp.max(acc)


def _sc_body(nb, x_hbm, rm_hbm, out_hbm, rmf_v, rmk_v, idx_v, rows_v, b_v,
             b2_v, sem):
    wid = lax.axis_index("s") * 2 + lax.axis_index("c")
    neg_inf = jnp.full((16,), -jnp.inf, jnp.float32)
    lane16 = lax.iota(jnp.int32, 16)

    def reselect(c, tkey, tval):
        t40 = _count40_key(lambda j: _sort_key(b_v[pl.ds(j * 16, 16)]),
                           _CAPV)
        t40val = _val_of_key(lambda j: b_v[pl.ds(j * 16, 16)], _CAPV, t40)
        t40v = _b16(t40)
        for j in range(_CAPV):
            b2_v[pl.ds(j * 16, 16)] = neg_inf

        def compact(j, off):
            v = b_v[pl.ds(j * 16, 16)]
            m = _sort_key(v) > t40v
            plsc.store_compressed(b2_v.at[pl.ds(off, 16)], v, mask=m)
            return off + jnp.sum(jnp.where(m, 1, 0))
        c2 = lax.fori_loop(0, _CAPV, compact, jnp.int32(0), unroll=False)
        for j in range(_CAPV):
            b_v[pl.ds(j * 16, 16)] = b2_v[pl.ds(j * 16, 16)]
        return c2, t40, t40val

    def keep(c, tkey, tval):
        return c, tkey, tval

    def process_batch(b):
        base_row = b * _NR
        pltpu.sync_copy(rm_hbm.at[b], rmf_v)
        for j in range(_NRV):
            rmk_v[pl.ds(j * 16, 16)] = _sort_key(rmf_v[pl.ds(j * 16, 16)])
        tkey = _count40_key(lambda j: rmk_v[pl.ds(j * 16, 16)], _NRV)
        tval = _val_of_key(lambda j: rmf_v[pl.ds(j * 16, 16)], _NRV, tkey)

        t1v = _b16(tkey)
        for j in range(_NRV + 1):
            idx_v[pl.ds(j * 16, 16)] = lane16 + _b16(j * 16 + base_row)
        nr_v = _b16(0)
        for j in range(_NRV):
            m = rmk_v[pl.ds(j * 16, 16)] >= t1v
            mi = jnp.where(m, 1, 0)
            pos = nr_v + plsc.cumsum(mi) - mi
            plsc.store_scatter(idx_v, [pos],
                               lane16 + _b16(j * 16 + base_row), mask=m)
            nr_v = nr_v + plsc.all_reduce_population_count(m)
        n_rows = jnp.max(nr_v)

        for j in range(_CAPV):
            b_v[pl.ds(j * 16, 16)] = neg_inf

        n_waves = (n_rows + (_WAVE - 1)) // _WAVE

        def wave(w, carry):
            c, tkey, tval = carry
            pltpu.async_copy(x_hbm.at[idx_v.at[pl.ds(w * _WAVE, _WAVE)]],
                             rows_v, sem).wait()
            rows_here = jnp.minimum(jnp.int32(_WAVE), n_rows - w * _WAVE)

            def row(r, carry):
                c, tkey, tval = carry
                c, tkey, tval = lax.cond(c > _CAP - _RL, reselect, keep,
                                         c, tkey, tval)
                tkeyv = _b16(tkey)

                def vreg(j, cv):
                    v = rows_v[r, pl.ds(j * 16, 16)]
                    m = _sort_key(v) > tkeyv
                    mi = jnp.where(m, 1, 0)
                    pos = cv + plsc.cumsum(mi) - mi
                    plsc.store_scatter(b_v, [pos], v, mask=m)
                    return cv + plsc.all_reduce_population_count(m)
                cv = lax.fori_loop(0, _RL // 16, vreg, _b16(c),
                                   unroll=True)
                return jnp.max(cv), tkey, tval
            return lax.fori_loop(0, rows_here, row, (c, tkey, tval),
                                 unroll=False)

        c, tkey, tval = lax.fori_loop(0, n_waves, wave,
                                      (jnp.int32(0), tkey, tval),
                                      unroll=False)
        c, tkey, tval = lax.cond(c > _OUT, reselect, keep, c, tkey, tval)
        tvalv = _b16(tval, jnp.float32)
        cv = _b16(c)
        kv = _b16(_K)
        for j in range(_OUT // 16):
            l = lane16 + _b16(j * 16)
            v = b_v[pl.ds(j * 16, 16)]
            v = jnp.where(l < cv, v, jnp.where(l < kv, tvalv, neg_inf))
            b2_v[pl.ds(j * 16, 16)] = v
        pltpu.sync_copy(b2_v.at[pl.ds(0, _OUT)], out_hbm.at[b])

    for bi in range(nb // _NW):
        process_batch(wid * (nb // _NW) + bi)


def _make_sc_kernel(nb):
    return functools.partial(
        pl.kernel,
        out_type=jax.ShapeDtypeStruct((nb, _OUT), jnp.float32),
        mesh=plsc.VectorSubcoreMesh(core_axis_name="c", subcore_axis_name="s",
                                    num_cores=2, num_subcores=16),
        compiler_params=pltpu.CompilerParams(needs_layout_passes=False),
        scratch_types=[
            pltpu.VMEM((_NR,), jnp.float32),
            pltpu.VMEM((_NR,), jnp.int32),
            pltpu.VMEM((_NR + 16,), jnp.int32),
            pltpu.VMEM((_WAVE, _RL), jnp.float32),
            pltpu.VMEM((_CAP,), jnp.float32),
            pltpu.VMEM((_CAP,), jnp.float32),
            pltpu.SemaphoreType.DMA,
        ],
    )(functools.partial(_sc_body, nb))


def _stage_c(cand_ref, ps_ref, out_ref):
    cand = cand_ref[...]
    key = _sort_key(cand)

    def bit_step(i, tu):
        candbit = tu | jnp.left_shift(jnp.int32(1), jnp.int32(31) - i)
        thr = candbit ^ jnp.int32(_INT_MIN)
        cnt = jnp.sum((key >= thr).astype(jnp.int32), axis=1, keepdims=True)
        return jnp.where(cnt >= _K, candbit, tu)

    tu = lax.fori_loop(0, 32, bit_step, jnp.zeros((_NB, 1), jnp.int32))
    t = tu ^ jnp.int32(_INT_MIN)
    gt = key > t
    cnt_gt = jnp.sum(gt.astype(jnp.float32), axis=1, keepdims=True)
    sum_gt = jnp.sum(jnp.where(gt, _softplus(cand), 0.0), axis=1,
                     keepdims=True)
    ns_b = sum_gt + (jnp.float32(_K) - cnt_gt) * _softplus(_key_to_val(t))
    ns = jnp.sum(ns_b) / jnp.float32(_NB * _K)
    ps = ps_ref[:, 0, 0:1] / ps_ref[:, 0, 1:2]
    out_ref[...] = jnp.broadcast_to(ns + ps, (_NB, 128))


def _half(ypred, truthMask, off, nb):
    x, rm, ps = pl.pallas_call(
        _stage_a,
        grid=(nb,),
        in_specs=[
            pl.BlockSpec((1, _NR, _RL), lambda i: (i + off, 0, 0)),
            pl.BlockSpec((1, _NR, _RL), lambda i: (i + off, 0, 0)),
        ],
        out_specs=[
            pl.BlockSpec((1, _NR, _RL), lambda i: (i, 0, 0)),
            pl.BlockSpec((1, 1, _NR), lambda i: (i, 0, 0)),
            pl.BlockSpec((1, 1, 128), lambda i: (i, 0, 0)),
        ],
        out_shape=[
            jax.ShapeDtypeStruct((nb, _NR, _RL), jnp.float32),
            jax.ShapeDtypeStruct((nb, 1, _NR), jnp.float32),
            jax.ShapeDtypeStruct((nb, 1, 128), jnp.float32),
        ],
    )(ypred, truthMask)
    cand = _make_sc_kernel(nb)(x.reshape(nb * _NR, _RL),
                               rm.reshape(nb, _NR))
    return cand, ps


def kernel(ypred, truthMask):
    nh = _NB // 2
    cand0, ps0 = _half(ypred, truthMask, 0, nh)
    cand1, ps1 = _half(ypred, truthMask, nh, nh)
    cand = jnp.concatenate([cand0, cand1])
    ps = jnp.concatenate([ps0, ps1])

    out = pl.pallas_call(
        _stage_c,
        in_specs=[
            pl.BlockSpec((_NB, _OUT), lambda: (0, 0)),
            pl.BlockSpec((_NB, 1, 128), lambda: (0, 0, 0)),
        ],
        out_specs=pl.BlockSpec((_NB, 128), lambda: (0, 0)),
        out_shape=jax.ShapeDtypeStruct((_NB, 128), jnp.float32),
    )(cand, ps)
    return out[:, 0]


# speedup vs baseline: 2.1273x; 1.0114x over previous
import functools

import jax
import jax.numpy as jnp
from jax import lax
from jax.experimental import pallas as pl
from jax.experimental.pallas import tpu as pltpu
from jax.experimental.pallas import tpu_sc as plsc

_K = 40
_INT_MIN = -(2**31)
_NB = 64
_NR = 384
_RL = 384
_NW = 32
_BPW = _NB // _NW
_WAVE = 48
_NRV = _NR // 16
_CAP = 512
_CAPV = _CAP // 16
_OUT = 128


def _softplus(x):
    return jnp.log(jnp.exp(x) + 1.0)


def _sort_key(x):
    u = lax.bitcast_convert_type(x, jnp.int32)
    return u ^ jnp.right_shift(u, 31) & jnp.int32(0x7FFFFFFF)


def _key_to_val(k):
    return lax.bitcast_convert_type(k ^ jnp.right_shift(k, 31)
                                    & jnp.int32(0x7FFFFFFF), jnp.float32)


def _stage_a(y_ref, m_ref, x_ref, rm_ref, ps_ref):
    y = y_ref[0]
    m = m_ref[0]
    ps_num = jnp.sum(jnp.where(m, _softplus(-y), 0.0))
    ps_den = jnp.sum(m.astype(jnp.float32))
    x = jnp.where(m, -jnp.inf, y)
    x_ref[0] = x
    rm_ref[0, 0] = jnp.max(x, axis=1)
    lane = lax.broadcasted_iota(jnp.int32, (1, 1, 128), 2)
    ps_ref[...] = jnp.where(lane == 0, ps_num, jnp.where(lane == 1, ps_den, 0.0))


def _b16(s, dtype=jnp.int32):
    return jnp.full((16,), s, dtype)


def _count40_key(read_vreg, nv):
    def bit_step(i, tu):
        cand = tu | jnp.left_shift(jnp.int32(1), jnp.int32(31) - i)
        thr = _b16(cand ^ jnp.int32(_INT_MIN))
        cnt_v = _b16(0)
        for j in range(nv):
            cnt_v = cnt_v + jnp.where(read_vreg(j) >= thr, 1, 0)
        cnt = jnp.sum(cnt_v)
        return jnp.where(cnt >= _K, cand, tu)

    tu = lax.fori_loop(0, 32, bit_step, jnp.int32(0), unroll=False)
    return tu ^ jnp.int32(_INT_MIN)


def _val_of_key(read_val, nv, tkey):
    tkeyv = _b16(tkey)
    acc = jnp.full((16,), -jnp.inf, jnp.float32)
    for j in range(nv):
        v = read_val(j)
        acc = jnp.where(_sort_key(v) == tkeyv, v, acc)
    return jnp.max(acc)


def _sc_body(nb, x_hbm, rm_hbm, out_hbm, rmf_v, rmk_v, idx_v, rows_v, b_v,
             b2_v, sem):
    wid = lax.axis_index("s") * 2 + lax.axis_index("c")
    neg_inf = jnp.full((16,), -jnp.inf, jnp.float32)
    lane16 = lax.iota(jnp.int32, 16)

    def reselect(c, tkey, tval):
        t40 = _count40_key(lambda j: _sort_key(b_v[pl.ds(j * 16, 16)]),
                           _CAPV)
        t40val = _val_of_key(lambda j: b_v[pl.ds(j * 16, 16)], _CAPV, t40)
        t40v = _b16(t40)
        for j in range(_CAPV):
            b2_v[pl.ds(j * 16, 16)] = neg_inf

        def compact(j, off):
            v = b_v[pl.ds(j * 16, 16)]
            m = _sort_key(v) > t40v
            plsc.store_compressed(b2_v.at[pl.ds(off, 16)], v, mask=m)
            return off + jnp.sum(jnp.where(m, 1, 0))
        c2 = lax.fori_loop(0, _CAPV, compact, jnp.int32(0), unroll=False)
        for j in range(_CAPV):
            b_v[pl.ds(j * 16, 16)] = b2_v[pl.ds(j * 16, 16)]
        return c2, t40, t40val

    def keep(c, tkey, tval):
        return c, tkey, tval

    def process_batch(b):
        base_row = b * _NR
        pltpu.sync_copy(rm_hbm.at[b], rmf_v)
        for j in range(_NRV):
            rmk_v[pl.ds(j * 16, 16)] = _sort_key(rmf_v[pl.ds(j * 16, 16)])
        tkey = _count40_key(lambda j: rmk_v[pl.ds(j * 16, 16)], _NRV)
        tval = _val_of_key(lambda j: rmf_v[pl.ds(j * 16, 16)], _NRV, tkey)

        t1v = _b16(tkey)
        for j in range(_NRV + 1):
            idx_v[pl.ds(j * 16, 16)] = lane16 + _b16(j * 16 + base_row)
        nr_v = _b16(0)
        for j in range(_NRV):
            m = rmk_v[pl.ds(j * 16, 16)] >= t1v
            mi = jnp.where(m, 1, 0)
            pos = nr_v + plsc.cumsum(mi) - mi
            plsc.store_scatter(idx_v, [pos],
                               lane16 + _b16(j * 16 + base_row), mask=m)
            nr_v = nr_v + plsc.all_reduce_population_count(m)
        n_rows = jnp.max(nr_v)

        for j in range(_CAPV):
            b_v[pl.ds(j * 16, 16)] = neg_inf

        n_waves = (n_rows + (_WAVE - 1)) // _WAVE

        def wave(w, carry):
            c, tkey, tval = carry
            pltpu.async_copy(x_hbm.at[idx_v.at[pl.ds(w * _WAVE, _WAVE)]],
                             rows_v, sem).wait()
            rows_here = jnp.minimum(jnp.int32(_WAVE), n_rows - w * _WAVE)

            def row(r, carry):
                c, tkey, tval = carry
                c, tkey, tval = lax.cond(c > _CAP - _RL, reselect, keep,
                                         c, tkey, tval)
                tkeyv = _b16(tkey)

                def vreg(j, cv):
                    v = rows_v[r, pl.ds(j * 16, 16)]
                    m = _sort_key(v) > tkeyv
                    mi = jnp.where(m, 1, 0)
                    pos = cv + plsc.cumsum(mi) - mi
                    plsc.store_scatter(b_v, [pos], v, mask=m)
                    return cv + plsc.all_reduce_population_count(m)
                cv = lax.fori_loop(0, _RL // 16, vreg, _b16(c),
                                   unroll=True)
                return jnp.max(cv), tkey, tval
            return lax.fori_loop(0, rows_here, row, (c, tkey, tval),
                                 unroll=False)

        c, tkey, tval = lax.fori_loop(0, n_waves, wave,
                                      (jnp.int32(0), tkey, tval),
                                      unroll=False)
        c, tkey, tval = lax.cond(c > _OUT, reselect, keep, c, tkey, tval)
        tvalv = _b16(tval, jnp.float32)
        cv = _b16(c)
        kv = _b16(_K)
        for j in range(_OUT // 16):
            l = lane16 + _b16(j * 16)
            v = b_v[pl.ds(j * 16, 16)]
            v = jnp.where(l < cv, v, jnp.where(l < kv, tvalv, neg_inf))
            b2_v[pl.ds(j * 16, 16)] = v
        pltpu.sync_copy(b2_v.at[pl.ds(0, _OUT)], out_hbm.at[b])

    for bi in range(nb // _NW):
        process_batch(wid * (nb // _NW) + bi)


def _make_sc_kernel(nb):
    return functools.partial(
        pl.kernel,
        out_type=jax.ShapeDtypeStruct((nb, _OUT), jnp.float32),
        mesh=plsc.VectorSubcoreMesh(core_axis_name="c", subcore_axis_name="s",
                                    num_cores=2, num_subcores=16),
        compiler_params=pltpu.CompilerParams(needs_layout_passes=False),
        scratch_types=[
            pltpu.VMEM((_NR,), jnp.float32),
            pltpu.VMEM((_NR,), jnp.int32),
            pltpu.VMEM((_NR + 16,), jnp.int32),
            pltpu.VMEM((_WAVE, _RL), jnp.float32),
            pltpu.VMEM((_CAP,), jnp.float32),
            pltpu.VMEM((_CAP,), jnp.float32),
            pltpu.SemaphoreType.DMA,
        ],
    )(functools.partial(_sc_body, nb))


def _stage_c(cand0_ref, cand1_ref, ps0_ref, ps1_ref, out_ref):
    cand = jnp.concatenate([cand0_ref[...], cand1_ref[...]])
    key = _sort_key(cand)

    def bit_step(i, tu):
        candbit = tu | jnp.left_shift(jnp.int32(1), jnp.int32(31) - i)
        thr = candbit ^ jnp.int32(_INT_MIN)
        cnt = jnp.sum((key >= thr).astype(jnp.int32), axis=1, keepdims=True)
        return jnp.where(cnt >= _K, candbit, tu)

    tu = lax.fori_loop(0, 32, bit_step, jnp.zeros((_NB, 1), jnp.int32))
    t = tu ^ jnp.int32(_INT_MIN)
    gt = key > t
    cnt_gt = jnp.sum(gt.astype(jnp.float32), axis=1, keepdims=True)
    sum_gt = jnp.sum(jnp.where(gt, _softplus(cand), 0.0), axis=1,
                     keepdims=True)
    ns_b = sum_gt + (jnp.float32(_K) - cnt_gt) * _softplus(_key_to_val(t))
    ns = jnp.sum(ns_b) / jnp.float32(_NB * _K)
    ps_all = jnp.concatenate([ps0_ref[...], ps1_ref[...]])
    ps = ps_all[:, 0, 0:1] / ps_all[:, 0, 1:2]
    out_ref[...] = jnp.broadcast_to(ns + ps, (_NB, 128))


def _half(ypred, truthMask, off, nb):
    x, rm, ps = pl.pallas_call(
        _stage_a,
        grid=(nb,),
        in_specs=[
            pl.BlockSpec((1, _NR, _RL), lambda i: (i + off, 0, 0)),
            pl.BlockSpec((1, _NR, _RL), lambda i: (i + off, 0, 0)),
        ],
        out_specs=[
            pl.BlockSpec((1, _NR, _RL), lambda i: (i, 0, 0)),
            pl.BlockSpec((1, 1, _NR), lambda i: (i, 0, 0)),
            pl.BlockSpec((1, 1, 128), lambda i: (i, 0, 0)),
        ],
        out_shape=[
            jax.ShapeDtypeStruct((nb, _NR, _RL), jnp.float32),
            jax.ShapeDtypeStruct((nb, 1, _NR), jnp.float32),
            jax.ShapeDtypeStruct((nb, 1, 128), jnp.float32),
        ],
    )(ypred, truthMask)
    cand = _make_sc_kernel(nb)(x.reshape(nb * _NR, _RL),
                               rm.reshape(nb, _NR))
    return cand, ps


def kernel(ypred, truthMask):
    nh = _NB // 2
    cand0, ps0 = _half(ypred, truthMask, 0, nh)
    cand1, ps1 = _half(ypred, truthMask, nh, nh)

    out = pl.pallas_call(
        _stage_c,
        in_specs=[
            pl.BlockSpec((nh, _OUT), lambda: (0, 0)),
            pl.BlockSpec((nh, _OUT), lambda: (0, 0)),
            pl.BlockSpec((nh, 1, 128), lambda: (0, 0, 0)),
            pl.BlockSpec((nh, 1, 128), lambda: (0, 0, 0)),
        ],
        out_specs=pl.BlockSpec((_NB, 128), lambda: (0, 0)),
        out_shape=jax.ShapeDtypeStruct((_NB, 128), jnp.float32),
    )(cand0, cand1, ps0, ps1)
    return out[:, 0]
